# baseline (speedup 1.0000x reference)
import functools

import jax
import jax.numpy as jnp
from jax import lax
from jax.experimental import pallas as pl
from jax.experimental.pallas import tpu as pltpu
from jax.experimental.pallas import tpu_sc as plsc

D = 64
DP = 128
IW = 128
N_ROW = 4
CHUNK = IW * N_ROW
SL = 16


def kernel(input_ids, weight):
    B, L = input_ids.shape
    V = weight.shape[0]
    btot = B * L
    info = plsc.get_sparse_core_info()
    nc = info.num_cores
    nw = nc * info.num_subcores
    assert btot % (nw * CHUNK) == 0
    b_per_w = btot // nw
    n_chunks = b_per_w // CHUNK
    assert n_chunks % 2 == 0 and n_chunks >= 4

    wlin = jnp.pad(weight, ((0, 0), (0, DP - D))).reshape(2 * V, D)

    ids2d = (input_ids.reshape(btot // IW, IW) * 2).astype(jnp.int32)
    ids2d = jnp.concatenate([ids2d, jnp.zeros((N_ROW, IW), jnp.int32)], axis=0)

    mesh = plsc.VectorSubcoreMesh(core_axis_name="c", subcore_axis_name="s")

    @functools.partial(
        pl.kernel,
        out_type=jax.ShapeDtypeStruct((btot, DP), jnp.float32),
        mesh=mesh,
        scratch_types=[
            pltpu.VMEM((N_ROW, IW), jnp.int32),
            pltpu.VMEM((N_ROW, IW), jnp.int32),
            pltpu.VMEM((CHUNK, D), jnp.float32),
            pltpu.VMEM((CHUNK, D), jnp.float32),
            pltpu.SemaphoreType.DMA,
            pltpu.SemaphoreType.DMA,
            pltpu.SemaphoreType.DMA,
            pltpu.SemaphoreType.DMA,
            pltpu.SemaphoreType.DMA,
            pltpu.SemaphoreType.DMA,
        ],
        compiler_params=pltpu.CompilerParams(use_tc_tiling_on_sc=False),
    )
    def gather_kernel(ids_hbm, table_hbm, out_hbm,
                      idx0, idx1, rows0, rows1,
                      sg0, sg1, so0, so1, si0, si1):
        wid = lax.axis_index("s") * nc + lax.axis_index("c")
        row_base = wid * (b_per_w // IW)

        idx = (idx0, idx1)
        rows = (rows0, rows1)
        sg = (sg0, sg1)
        so = (so0, so1)
        si = (si0, si1)

        def idx_copy(i, b):
            row_off = row_base + i * N_ROW
            return pltpu.make_async_copy(
                ids_hbm.at[pl.ds(row_off, N_ROW)], idx[b], si[b])

        def gather_copies(b):
            cs = []
            for j in range(N_ROW):
                for k in range(IW // SL):
                    s = j * (IW // SL) + k
                    cs.append(pltpu.make_async_copy(
                        table_hbm.at[idx[b].at[j, pl.ds(k * SL, SL)]],
                        rows[b].at[pl.ds(s * SL, SL)],
                        sg[b]))
            return cs

        def out_copy(i, b):
            row_off = row_base + i * N_ROW
            return pltpu.make_async_copy(
                rows[b],
                out_hbm.at[pl.ds(row_off * IW, CHUNK), pl.ds(0, D)],
                so[b])

        pltpu.sync_copy(ids_hbm.at[pl.ds(row_base, N_ROW)], idx0)
        for c in gather_copies(0):
            c.start()
        pltpu.sync_copy(ids_hbm.at[pl.ds(row_base + N_ROW, N_ROW)], idx1)
        for c in gather_copies(0):
            c.wait()
        out_copy(0, 0).start()
        idx_copy(2, 0).start()
        for c in gather_copies(1):
            c.start()

        def pair(g, carry):
            i0 = 2 * g
            for i, b in ((i0, 0), (i0 + 1, 1)):
                ob = 1 - b
                for c in gather_copies(ob):
                    c.wait()
                out_copy(i - 1, ob).start()
                idx_copy(i + 1, ob).start()
                out_copy(i - 2, b).wait()
                idx_copy(i, b).wait()
                for c in gather_copies(b):
                    c.start()
            return carry

        lax.fori_loop(1, n_chunks // 2, pair, 0)

        last = n_chunks - 1
        for c in gather_copies(1):
            c.wait()
        out_copy(last, 1).start()
        out_copy(last - 1, 0).wait()
        idx_copy(n_chunks, 0).wait()
        out_copy(last, 1).wait()

    out = gather_kernel(ids2d, wlin)
    return out.reshape(B, L, DP)[:, :, :D]
